# Initial kernel scaffold; baseline (speedup 1.0000x reference)
import jax
import jax.numpy as jnp
from jax.experimental import pallas as pl


def kernel(indices, emb_table, pw, pb, v1w, v1b, v2w, v2b, v3w, v3b):
    raise NotImplementedError("write your pallas kernel here")


# speedup vs baseline: 9.4524x; 9.4524x over previous
import functools

import jax
import jax.numpy as jnp
from jax import lax
from jax.experimental import pallas as pl
from jax.experimental.pallas import tpu as pltpu
from jax.experimental.pallas import tpu_sc as plsc

NC, NS, LANES = 2, 16, 16
NW = NC * NS

B, L, D = 16384, 200, 128
SPW = B // NW
IDX_CHUNK = 100
NCH = L // IDX_CHUNK
BLK = 16
NBLK = SPW // BLK


def _bag_body(idx_hbm, tab_hbm, x_hbm, idx_v, buf, acc_v, sem):
    cid = lax.axis_index("c")
    sid = lax.axis_index("s")
    wid = sid * NC + cid

    def blk_body(b, carry):
        gblk = wid * NBLK + b
        pltpu.sync_copy(idx_hbm.at[gblk], idx_v)

        def s_body(s, carry2):
            cp0 = pltpu.async_copy(tab_hbm.at[idx_v.at[2 * s]], buf.at[0], sem)
            cp1 = pltpu.async_copy(tab_hbm.at[idx_v.at[2 * s + 1]], buf.at[1], sem)
            cp0.wait()
            cp1.wait()

            def r_body(r, accs):
                out = []
                for k in range(D // LANES):
                    v0 = buf[0, r, pl.ds(k * LANES, LANES)]
                    v1 = buf[1, r, pl.ds(k * LANES, LANES)]
                    out.append(accs[k] + v0 + v1)
                return tuple(out)

            init = tuple(jnp.zeros((LANES,), jnp.float32) for _ in range(D // LANES))
            accs = lax.fori_loop(0, IDX_CHUNK, r_body, init)
            for k in range(D // LANES):
                acc_v[s, pl.ds(k * LANES, LANES)] = accs[k]
            return carry2

        lax.fori_loop(0, BLK, s_body, 0)
        pltpu.sync_copy(acc_v, x_hbm.at[pl.ds(gblk * BLK, BLK)])
        return carry

    lax.fori_loop(0, NBLK, blk_body, 0)


def _embedding_bag(indices, emb_table):
    idx3 = indices.astype(jnp.int32).reshape(B // BLK, BLK * NCH, IDX_CHUNK)
    mesh = plsc.VectorSubcoreMesh(
        core_axis_name="c", subcore_axis_name="s", num_cores=NC, num_subcores=NS
    )
    return pl.kernel(
        _bag_body,
        out_type=jax.ShapeDtypeStruct((B, D), jnp.float32),
        mesh=mesh,
        scratch_types=[
            pltpu.VMEM((BLK * NCH, IDX_CHUNK), jnp.int32),
            pltpu.VMEM((NCH, IDX_CHUNK, D), jnp.float32),
            pltpu.VMEM((BLK, D), jnp.float32),
            pltpu.SemaphoreType.DMA,
        ],
    )(idx3, emb_table)


TB = 1024


def _head_body(x_ref, pwt_ref, pb_ref, w1_ref, b1_ref, w2_ref, b2_ref,
               w3_ref, b3_ref, pol_ref, val_ref):
    x = x_ref[...]
    hi = lax.Precision.HIGHEST
    pol_ref[...] = (
        lax.dot_general(x, pwt_ref[...], (((1,), (0,)), ((), ())), precision=hi)
        + pb_ref[...]
    )
    h = jnp.clip(
        lax.dot_general(x, w1_ref[...], (((1,), (0,)), ((), ())), precision=hi)
        + b1_ref[...], 0.0, 1.0)
    h = jnp.clip(
        lax.dot_general(h, w2_ref[...], (((1,), (0,)), ((), ())), precision=hi)
        + b2_ref[...], 0.0, 1.0)
    val_ref[...] = jnp.tanh(
        lax.dot_general(h, w3_ref[...], (((1,), (0,)), ((), ())), precision=hi)
        + b3_ref[...])


def _heads(x, pw, pb, v1w, v1b, v2w, v2b, v3w, v3b):
    np_ = pw.shape[0]
    full = lambda shape: pl.BlockSpec(shape, lambda i: (0, 0))
    return pl.pallas_call(
        _head_body,
        grid=(B // TB,),
        in_specs=[
            pl.BlockSpec((TB, D), lambda i: (i, 0)),
            full((D, np_)),
            full((1, np_)),
            full((D, 32)),
            full((1, 32)),
            full((32, 32)),
            full((1, 32)),
            full((32, 1)),
            full((1, 1)),
        ],
        out_specs=[
            pl.BlockSpec((TB, np_), lambda i: (i, 0)),
            pl.BlockSpec((TB, 1), lambda i: (i, 0)),
        ],
        out_shape=[
            jax.ShapeDtypeStruct((B, np_), jnp.float32),
            jax.ShapeDtypeStruct((B, 1), jnp.float32),
        ],
    )(
        x, pw.T, pb.reshape(1, np_), v1w.T, v1b.reshape(1, 32),
        v2w.T, v2b.reshape(1, 32), v3w.T, v3b.reshape(1, 1),
    )


def kernel(indices, emb_table, pw, pb, v1w, v1b, v2w, v2b, v3w, v3b):
    x = _embedding_bag(indices, emb_table)
    pol, val = _heads(x, pw, pb, v1w, v1b, v2w, v2b, v3w, v3b)
    return (pol, val)


# speedup vs baseline: 16.8426x; 1.7818x over previous
import functools

import jax
import jax.numpy as jnp
from jax import lax
from jax.experimental import pallas as pl
from jax.experimental.pallas import tpu as pltpu
from jax.experimental.pallas import tpu_sc as plsc

NC, NS, LANES = 2, 16, 16
NW = NC * NS

B, L, D = 16384, 200, 128
SPW = B // NW
IDX_CHUNK = 100
NCH = L // IDX_CHUNK
BLK = 32
NBLK = SPW // BLK
NSLOT = 2


def _bag_body(idx_hbm, tab_hbm, x_hbm, idx_v, buf, acc_v, sem0, sem1):
    cid = lax.axis_index("c")
    sid = lax.axis_index("s")
    wid = sid * NC + cid
    sems = (sem0, sem1)

    def fire(t, slot):
        row0 = lax.rem(t, 2 * BLK) * NCH
        for c in range(NCH):
            pltpu.async_copy(tab_hbm.at[idx_v.at[row0 + c]],
                             buf.at[slot].at[c], sems[slot])

    def drain(slot):
        for c in range(NCH):
            pltpu.make_async_copy(tab_hbm.at[idx_v.at[0]],
                                  buf.at[slot].at[c], sems[slot]).wait()

    def stage_idx(nb):
        bank = lax.rem(nb, 2)
        off = pl.multiple_of(bank * BLK * NCH, BLK * NCH)
        pltpu.sync_copy(idx_hbm.at[wid * NBLK + nb],
                        idx_v.at[pl.ds(off, BLK * NCH)])

    def accum(s, slot):
        def r_body(r, accs):
            out = list(accs)
            for c in range(NCH):
                for k in range(D // LANES):
                    v = buf[slot, c, r, pl.ds(k * LANES, LANES)]
                    out[k] = out[k] + v
            return tuple(out)

        init = tuple(jnp.zeros((LANES,), jnp.float32) for _ in range(D // LANES))
        accs = lax.fori_loop(0, IDX_CHUNK, r_body, init)
        arow = lax.rem(s, BLK)
        for k in range(D // LANES):
            acc_v[arow, pl.ds(k * LANES, LANES)] = accs[k]

    stage_idx(0)
    fire(0, 0)
    fire(1, 1)

    def g_body(g2, carry):
        g = g2 * NSLOT
        for b in range(NSLOT):
            s = g + b
            t = s + NSLOT
            drain(b)
            accum(s, b)

            @pl.when(jnp.logical_and(lax.rem(t, BLK) == 0, t < SPW))
            def _():
                stage_idx(t // BLK)

            @pl.when(t < SPW)
            def _():
                fire(t, b)

        s_last = g + NSLOT - 1

        @pl.when(lax.rem(s_last, BLK) == BLK - 1)
        def _():
            base = pl.multiple_of(wid * SPW + s_last - (BLK - 1), BLK)
            pltpu.sync_copy(acc_v, x_hbm.at[pl.ds(base, BLK)])

        return carry

    lax.fori_loop(0, SPW // NSLOT, g_body, 0)


def _embedding_bag(indices, emb_table):
    idx3 = indices.astype(jnp.int32).reshape(B // BLK, BLK * NCH, IDX_CHUNK)
    mesh = plsc.VectorSubcoreMesh(
        core_axis_name="c", subcore_axis_name="s", num_cores=NC, num_subcores=NS
    )
    return pl.kernel(
        _bag_body,
        out_type=jax.ShapeDtypeStruct((B, D), jnp.float32),
        mesh=mesh,
        scratch_types=[
            pltpu.VMEM((2 * BLK * NCH, IDX_CHUNK), jnp.int32),
            pltpu.VMEM((NSLOT, NCH, IDX_CHUNK, D), jnp.float32),
            pltpu.VMEM((BLK, D), jnp.float32),
            pltpu.SemaphoreType.DMA,
            pltpu.SemaphoreType.DMA,
        ],
    )(idx3, emb_table)


TB = 1024


def _head_body(x_ref, pwt_ref, pb_ref, w1_ref, b1_ref, w2_ref, b2_ref,
               w3_ref, b3_ref, pol_ref, val_ref):
    x = x_ref[...]
    hi = lax.Precision.HIGHEST
    pol_ref[...] = (
        lax.dot_general(x, pwt_ref[...], (((1,), (0,)), ((), ())), precision=hi)
        + pb_ref[...]
    )
    h = jnp.clip(
        lax.dot_general(x, w1_ref[...], (((1,), (0,)), ((), ())), precision=hi)
        + b1_ref[...], 0.0, 1.0)
    h = jnp.clip(
        lax.dot_general(h, w2_ref[...], (((1,), (0,)), ((), ())), precision=hi)
        + b2_ref[...], 0.0, 1.0)
    val_ref[...] = jnp.tanh(
        lax.dot_general(h, w3_ref[...], (((1,), (0,)), ((), ())), precision=hi)
        + b3_ref[...])


def _heads(x, pw, pb, v1w, v1b, v2w, v2b, v3w, v3b):
    np_ = pw.shape[0]
    full = lambda shape: pl.BlockSpec(shape, lambda i: (0, 0))
    return pl.pallas_call(
        _head_body,
        grid=(B // TB,),
        in_specs=[
            pl.BlockSpec((TB, D), lambda i: (i, 0)),
            full((D, np_)),
            full((1, np_)),
            full((D, 32)),
            full((1, 32)),
            full((32, 32)),
            full((1, 32)),
            full((32, 1)),
            full((1, 1)),
        ],
        out_specs=[
            pl.BlockSpec((TB, np_), lambda i: (i, 0)),
            pl.BlockSpec((TB, 1), lambda i: (i, 0)),
        ],
        out_shape=[
            jax.ShapeDtypeStruct((B, np_), jnp.float32),
            jax.ShapeDtypeStruct((B, 1), jnp.float32),
        ],
    )(
        x, pw.T, pb.reshape(1, np_), v1w.T, v1b.reshape(1, 32),
        v2w.T, v2b.reshape(1, 32), v3w.T, v3b.reshape(1, 1),
    )


def kernel(indices, emb_table, pw, pb, v1w, v1b, v2w, v2b, v3w, v3b):
    x = _embedding_bag(indices, emb_table)
    pol, val = _heads(x, pw, pb, v1w, v1b, v2w, v2b, v3w, v3b)
    return (pol, val)
